# baseline (speedup 1.0000x reference)
import functools

import jax
import jax.numpy as jnp
from jax import lax
from jax.experimental import pallas as pl
from jax.experimental.pallas import tpu as pltpu
from jax.experimental.pallas import tpu_sc as plsc

_NC = 2
_NS = 16
_NW = _NC * _NS
_K = 26
_H = 128
_L = 16
_C = 4


def _make_sc_embed(b_static, b_dynamic, vocab):
  mesh = plsc.VectorSubcoreMesh(
      core_axis_name="c", subcore_axis_name="s",
      num_cores=_NC, num_subcores=_NS)

  @functools.partial(
      pl.kernel,
      out_type=(jax.ShapeDtypeStruct((b_static, _H), jnp.float32),
                jax.ShapeDtypeStruct((b_dynamic, _H), jnp.float32)),
      mesh=mesh,
      scratch_types=[
          [pltpu.VMEM((_C * _K,), jnp.int32)] * 4,
          [pltpu.VMEM((_C * _K,), jnp.float32)] * 4,
          [pltpu.VMEM((_C * _K, _H), jnp.float32)] * 4,
          [pltpu.VMEM((_C, _H), jnp.float32)] * 2,
          [pltpu.SemaphoreType.DMA] * 4,
          [pltpu.SemaphoreType.DMA] * 4,
          [pltpu.SemaphoreType.DMA] * 4,
          [pltpu.SemaphoreType.DMA] * 2,
      ],
  )
  def sc_embed(sids_hbm, sw_hbm, dids_hbm, dw_hbm, table_hbm,
               outs_hbm, outd_hbm, idx_v, w_v, rows_v, out_v,
               si, sw, sg, so):
    wid = lax.axis_index("s") * _NC + lax.axis_index("c")

    def run_phase(ids_hbm, wts_hbm, out_hbm, per_w):
      iters = per_w // _C
      base = wid * per_w

      def stage(it, slot, sync=False):
        bag0 = base + it * _C
        ci = pltpu.make_async_copy(
            ids_hbm.at[pl.ds(bag0 * _K, _C * _K)], idx_v[slot], si[slot])
        cw = pltpu.make_async_copy(
            wts_hbm.at[pl.ds(bag0 * _K, _C * _K)], w_v[slot], sw[slot])
        ci.start()
        cw.start()
        if sync:
          ci.wait()
          cw.wait()

      def stage_wait(slot):
        pltpu.make_async_copy(
            ids_hbm.at[pl.ds(0, _C * _K)], idx_v[slot], si[slot]).wait()
        pltpu.make_async_copy(
            wts_hbm.at[pl.ds(0, _C * _K)], w_v[slot], sw[slot]).wait()

      def gather(s4):
        pltpu.make_async_copy(
            table_hbm.at[idx_v[s4]], rows_v[s4], sg[s4]).start()

      def gather_wait(s4):
        pltpu.make_async_copy(
            table_hbm.at[idx_v[s4]], rows_v[s4], sg[s4]).wait()

      def out_start(it, slot):
        bag0 = base + it * _C
        pltpu.make_async_copy(
            out_v[slot], out_hbm.at[pl.ds(bag0, _C)], so[slot]).start()

      def out_wait(slot):
        pltpu.make_async_copy(
            out_v[slot], out_hbm.at[pl.ds(base, _C)], so[slot]).wait()

      def compute(slot, oslot):
        def bag(i, carry):
          wv0 = w_v[slot][pl.ds(i * _K, _L)]
          wv1 = w_v[slot][pl.ds(i * _K + 10, _L)]
          accs = [jnp.zeros((_L,), jnp.float32) for _ in range(_H // _L)]
          for j in range(_K):
            w = wv0[j] if j < _L else wv1[j - 10]
            for h in range(_H // _L):
              accs[h] = accs[h] + w * rows_v[slot][i * _K + j,
                                                   pl.ds(h * _L, _L)]
          for h in range(_H // _L):
            out_v[oslot][i, pl.ds(h * _L, _L)] = accs[h]
          return carry

        lax.fori_loop(0, _C, bag, 0)

      stage(0, 0, sync=True)
      gather(0)
      stage(1, 1, sync=True)
      gather(1)
      stage(2, 2)

      def step(it4, carry):
        for b in range(4):
          it = it4 * 4 + b
          s2 = b % 2

          @pl.when(it < iters - 2)
          def _fire_next():
            stage_wait((b + 2) % 4)
            gather((b + 2) % 4)

          gather_wait(b)

          @pl.when(it < iters - 3)
          def _stage_ahead():
            stage(it + 3, (b + 3) % 4)

          @pl.when(it >= 2)
          def _drain_out():
            out_wait(s2)

          compute(b, s2)
          out_start(it, s2)
        return carry

      lax.fori_loop(0, iters // 4, step, 0)
      out_wait(0)
      out_wait(1)

    run_phase(sids_hbm, sw_hbm, outs_hbm, b_static // _NW)
    run_phase(dids_hbm, dw_hbm, outd_hbm, b_dynamic // _NW)

  return sc_embed


def kernel(static_ids, static_values, static_values_mask, dynamic_ids,
           dynamic_values, dynamic_values_mask, event_mask, table):
  b, ns = static_ids.shape
  bd, s, m = dynamic_ids.shape
  v, h = table.shape
  assert ns == _K and m == _K and h == _H

  dids_m = jnp.where(event_mask[:, :, None], dynamic_ids, 0)
  dw = jnp.where(dynamic_values_mask, dynamic_values, 1.0)
  dw = dw * (dids_m != 0).astype(jnp.float32)
  sw = jnp.where(static_values_mask, static_values, 1.0)
  sw = sw * (static_ids != 0).astype(jnp.float32)

  out_s, out_d = _make_sc_embed(b, bd * s, v)(
      static_ids.astype(jnp.int32).reshape(-1),
      sw.reshape(-1),
      dids_m.astype(jnp.int32).reshape(-1),
      dw.reshape(-1),
      table)
  return (out_s, out_d.reshape(bd, s, _H))


# speedup vs baseline: 1.0028x; 1.0028x over previous
import functools

import jax
import jax.numpy as jnp
from jax import lax
from jax.experimental import pallas as pl
from jax.experimental.pallas import tpu as pltpu
from jax.experimental.pallas import tpu_sc as plsc

_NC = 2
_NS = 16
_NW = _NC * _NS
_K = 26
_H = 128
_L = 16
_C = 4
_KP = 32


def _make_sc_embed(b_static, b_dynamic, vocab):
  mesh = plsc.VectorSubcoreMesh(
      core_axis_name="c", subcore_axis_name="s",
      num_cores=_NC, num_subcores=_NS)

  @functools.partial(
      pl.kernel,
      out_type=(jax.ShapeDtypeStruct((b_static, _H), jnp.float32),
                jax.ShapeDtypeStruct((b_dynamic, _H), jnp.float32)),
      mesh=mesh,
      scratch_types=[
          [pltpu.VMEM((_C * _K,), jnp.int32)] * 4,
          [pltpu.VMEM((_C * _KP,), jnp.float32)] * 4,
          [pltpu.VMEM((_C * _K, _H), jnp.float32)] * 4,
          [pltpu.VMEM((_C, _H), jnp.float32)] * 2,
          [pltpu.SemaphoreType.DMA] * 4,
          [pltpu.SemaphoreType.DMA] * 4,
          [pltpu.SemaphoreType.DMA] * 4,
          [pltpu.SemaphoreType.DMA] * 2,
      ],
  )
  def sc_embed(sids_hbm, sw_hbm, dids_hbm, dw_hbm, table_hbm,
               outs_hbm, outd_hbm, idx_v, w_v, rows_v, out_v,
               si, sw, sg, so):
    wid = lax.axis_index("s") * _NC + lax.axis_index("c")

    def run_phase(ids_hbm, wts_hbm, out_hbm, per_w):
      iters = per_w // _C
      base = wid * per_w

      def stage(it, slot, sync=False):
        bag0 = base + it * _C
        ci = pltpu.make_async_copy(
            ids_hbm.at[pl.ds(bag0 * _K, _C * _K)], idx_v[slot], si[slot])
        cw = pltpu.make_async_copy(
            wts_hbm.at[pl.ds(bag0 * _KP, _C * _KP)], w_v[slot], sw[slot])
        ci.start()
        cw.start()
        if sync:
          ci.wait()
          cw.wait()

      def stage_wait(slot):
        pltpu.make_async_copy(
            ids_hbm.at[pl.ds(0, _C * _K)], idx_v[slot], si[slot]).wait()
        pltpu.make_async_copy(
            wts_hbm.at[pl.ds(0, _C * _KP)], w_v[slot], sw[slot]).wait()

      def gather(s4):
        pltpu.make_async_copy(
            table_hbm.at[idx_v[s4]], rows_v[s4], sg[s4]).start()

      def gather_wait(s4):
        pltpu.make_async_copy(
            table_hbm.at[idx_v[s4]], rows_v[s4], sg[s4]).wait()

      def out_start(it, slot):
        bag0 = base + it * _C
        pltpu.make_async_copy(
            out_v[slot], out_hbm.at[pl.ds(bag0, _C)], so[slot]).start()

      def out_wait(slot):
        pltpu.make_async_copy(
            out_v[slot], out_hbm.at[pl.ds(base, _C)], so[slot]).wait()

      def compute(slot, oslot):
        def bag(i, carry):
          wv0 = w_v[slot][pl.ds(i * _KP, _L)]
          wv1 = w_v[slot][pl.ds(i * _KP + _L, _L)]
          accs = [jnp.zeros((_L,), jnp.float32) for _ in range(_H // _L)]
          for j in range(_K):
            w = wv0[j] if j < _L else wv1[j - _L]
            for h in range(_H // _L):
              accs[h] = accs[h] + w * rows_v[slot][i * _K + j,
                                                   pl.ds(h * _L, _L)]
          for h in range(_H // _L):
            out_v[oslot][i, pl.ds(h * _L, _L)] = accs[h]
          return carry

        lax.fori_loop(0, _C, bag, 0)

      stage(0, 0, sync=True)
      gather(0)
      stage(1, 1, sync=True)
      gather(1)
      stage(2, 2)

      def step(it4, carry):
        for b in range(4):
          it = it4 * 4 + b
          s2 = b % 2

          @pl.when(it < iters - 2)
          def _fire_next():
            stage_wait((b + 2) % 4)
            gather((b + 2) % 4)

          gather_wait(b)

          @pl.when(it < iters - 3)
          def _stage_ahead():
            stage(it + 3, (b + 3) % 4)

          @pl.when(it >= 2)
          def _drain_out():
            out_wait(s2)

          compute(b, s2)
          out_start(it, s2)
        return carry

      lax.fori_loop(0, iters // 4, step, 0)
      out_wait(0)
      out_wait(1)

    run_phase(sids_hbm, sw_hbm, outs_hbm, b_static // _NW)
    run_phase(dids_hbm, dw_hbm, outd_hbm, b_dynamic // _NW)

  return sc_embed


def kernel(static_ids, static_values, static_values_mask, dynamic_ids,
           dynamic_values, dynamic_values_mask, event_mask, table):
  b, ns = static_ids.shape
  bd, s, m = dynamic_ids.shape
  v, h = table.shape
  assert ns == _K and m == _K and h == _H

  dids_m = jnp.where(event_mask[:, :, None], dynamic_ids, 0)
  dw = jnp.where(dynamic_values_mask, dynamic_values, 1.0)
  dw = dw * (dids_m != 0).astype(jnp.float32)
  sw = jnp.where(static_values_mask, static_values, 1.0)
  sw = sw * (static_ids != 0).astype(jnp.float32)
  sw = jnp.pad(sw, ((0, 0), (0, _KP - _K)))
  dw = jnp.pad(dw, ((0, 0), (0, 0), (0, _KP - _K)))

  out_s, out_d = _make_sc_embed(b, bd * s, v)(
      static_ids.astype(jnp.int32).reshape(-1),
      sw.reshape(-1),
      dids_m.astype(jnp.int32).reshape(-1),
      dw.reshape(-1),
      table)
  return (out_s, out_d.reshape(bd, s, _H))


# speedup vs baseline: 47.1350x; 47.0043x over previous
import functools

import jax
import jax.numpy as jnp
from jax import lax
from jax.experimental import pallas as pl
from jax.experimental.pallas import tpu as pltpu
from jax.experimental.pallas import tpu_sc as plsc

_NC = 2
_NS = 16
_NW = _NC * _NS
_K = 26
_H = 128
_L = 16
_C = 4
_KP = 32


def _make_sc_embed(b_static, b_dynamic, vocab):
  mesh = plsc.VectorSubcoreMesh(
      core_axis_name="c", subcore_axis_name="s",
      num_cores=_NC, num_subcores=_NS)

  @functools.partial(
      pl.kernel,
      out_type=(jax.ShapeDtypeStruct((b_static, _H), jnp.float32),
                jax.ShapeDtypeStruct((b_dynamic, _H), jnp.float32)),
      mesh=mesh,
      scratch_types=[
          [pltpu.VMEM((_C * _K,), jnp.int32)] * 4,
          [pltpu.VMEM((_C * _KP,), jnp.float32)] * 4,
          [pltpu.VMEM((_C * _K, _H), jnp.float32)] * 4,
          [pltpu.VMEM((_C, _H), jnp.float32)] * 2,
          [pltpu.SemaphoreType.DMA] * 4,
          [pltpu.SemaphoreType.DMA] * 4,
          [pltpu.SemaphoreType.DMA] * 4,
          [pltpu.SemaphoreType.DMA] * 2,
      ],
  )
  def sc_embed(sids_hbm, sw_hbm, dids_hbm, dw_hbm, table_hbm,
               outs_hbm, outd_hbm, idx_v, w_v, rows_v, out_v,
               si, sw, sg, so):
    wid = lax.axis_index("s") * _NC + lax.axis_index("c")

    def run_phase(ids_hbm, wts_hbm, out_hbm, per_w):
      iters = per_w // _C
      base = wid * per_w

      def stage(it, slot, sync=False):
        bag0 = base + it * _C
        ci = pltpu.make_async_copy(
            ids_hbm.at[pl.ds(bag0 * _K, _C * _K)], idx_v[slot], si[slot])
        cw = pltpu.make_async_copy(
            wts_hbm.at[pl.ds(bag0 * _KP, _C * _KP)], w_v[slot], sw[slot])
        ci.start()
        cw.start()
        if sync:
          ci.wait()
          cw.wait()

      def stage_wait(slot):
        pltpu.make_async_copy(
            ids_hbm.at[pl.ds(0, _C * _K)], idx_v[slot], si[slot]).wait()
        pltpu.make_async_copy(
            wts_hbm.at[pl.ds(0, _C * _KP)], w_v[slot], sw[slot]).wait()

      def gather(s4):
        pltpu.make_async_copy(
            table_hbm.at[idx_v[s4]], rows_v[s4], sg[s4]).start()

      def gather_wait(s4):
        pltpu.make_async_copy(
            table_hbm.at[idx_v[s4]], rows_v[s4], sg[s4]).wait()

      def out_start(it, slot):
        bag0 = base + it * _C
        pltpu.make_async_copy(
            out_v[slot], out_hbm.at[pl.ds(bag0, _C)], so[slot]).start()

      def out_wait(slot):
        pltpu.make_async_copy(
            out_v[slot], out_hbm.at[pl.ds(base, _C)], so[slot]).wait()

      def compute(slot, oslot):
        def bag(i, carry):
          wv0 = w_v[slot][pl.ds(i * _KP, _L)]
          wv1 = w_v[slot][pl.ds(i * _KP + _L, _L)]
          accs = [jnp.zeros((_L,), jnp.float32) for _ in range(_H // _L)]
          for j in range(_K):
            w = wv0[j] if j < _L else wv1[j - _L]
            for h in range(_H // _L):
              accs[h] = accs[h] + w * rows_v[slot][i * _K + j,
                                                   pl.ds(h * _L, _L)]
          for h in range(_H // _L):
            out_v[oslot][i, pl.ds(h * _L, _L)] = accs[h]
          return carry

        lax.fori_loop(0, _C, bag, 0)

      stage(0, 0, sync=True)
      gather(0)
      stage(1, 1, sync=True)
      gather(1)
      stage(2, 2)

      def step(it4, carry):
        for b in range(4):
          it = it4 * 4 + b
          s2 = b % 2

          @pl.when(it < iters - 2)
          def _fire_next():
            stage_wait((b + 2) % 4)
            gather((b + 2) % 4)

          gather_wait(b)

          @pl.when(it < iters - 3)
          def _stage_ahead():
            stage(it + 3, (b + 3) % 4)

          @pl.when(it >= 2)
          def _drain_out():
            out_wait(s2)

          compute(b, s2)
          out_start(it, s2)
        return carry

      lax.fori_loop(0, iters // 4, step, 0)
      out_wait(0)
      out_wait(1)

    run_phase(sids_hbm, sw_hbm, outs_hbm, b_static // _NW)
    run_phase(dids_hbm, dw_hbm, outd_hbm, b_dynamic // _NW)

  return sc_embed


def kernel(static_ids, static_values, static_values_mask, dynamic_ids,
           dynamic_values, dynamic_values_mask, event_mask, table):
  b, ns = static_ids.shape
  bd, s, m = dynamic_ids.shape
  v, h = table.shape
  assert ns == _K and m == _K and h == _H

  sw = jnp.where(static_values_mask, static_values, 1.0)
  sw = sw * (static_ids != 0).astype(jnp.float32)
  dw = jnp.where(dynamic_values_mask, dynamic_values, 1.0)
  dw = dw * (dynamic_ids != 0).astype(jnp.float32)
  dw = dw * event_mask[:, :, None].astype(jnp.float32)
  sw = jnp.pad(sw, ((0, 0), (0, _KP - _K)))
  dw = jnp.pad(dw, ((0, 0), (0, 0), (0, _KP - _K)))

  out_s, out_d = _make_sc_embed(b, bd * s, v)(
      static_ids.astype(jnp.int32).reshape(-1),
      sw.reshape(-1),
      dynamic_ids.astype(jnp.int32).reshape(-1),
      dw.reshape(-1),
      table)
  return (out_s, out_d.reshape(bd, s, _H))


# speedup vs baseline: 51.4301x; 1.0911x over previous
import functools

import jax
import jax.numpy as jnp
from jax import lax
from jax.experimental import pallas as pl
from jax.experimental.pallas import tpu as pltpu
from jax.experimental.pallas import tpu_sc as plsc

_NC = 2
_NS = 16
_NW = _NC * _NS
_K = 26
_H = 128
_L = 16
_C = 8
_KP = 32


def _make_sc_embed(b_static, b_dynamic, vocab):
  mesh = plsc.VectorSubcoreMesh(
      core_axis_name="c", subcore_axis_name="s",
      num_cores=_NC, num_subcores=_NS)

  @functools.partial(
      pl.kernel,
      out_type=(jax.ShapeDtypeStruct((b_static, _H), jnp.float32),
                jax.ShapeDtypeStruct((b_dynamic, _H), jnp.float32)),
      mesh=mesh,
      scratch_types=[
          [pltpu.VMEM((_C * _K,), jnp.int32)] * 4,
          [pltpu.VMEM((_C * _KP,), jnp.float32)] * 4,
          [pltpu.VMEM((_C * _K, _H), jnp.float32)] * 4,
          [pltpu.VMEM((_C, _H), jnp.float32)] * 2,
          [pltpu.SemaphoreType.DMA] * 4,
          [pltpu.SemaphoreType.DMA] * 4,
          [pltpu.SemaphoreType.DMA] * 4,
          [pltpu.SemaphoreType.DMA] * 2,
      ],
  )
  def sc_embed(sids_hbm, sw_hbm, dids_hbm, dw_hbm, table_hbm,
               outs_hbm, outd_hbm, idx_v, w_v, rows_v, out_v,
               si, sw, sg, so):
    wid = lax.axis_index("s") * _NC + lax.axis_index("c")

    def run_phase(ids_hbm, wts_hbm, out_hbm, per_w):
      iters = per_w // _C
      base = wid * per_w

      def stage(it, slot, sync=False):
        bag0 = base + it * _C
        ci = pltpu.make_async_copy(
            ids_hbm.at[pl.ds(bag0 * _K, _C * _K)], idx_v[slot], si[slot])
        cw = pltpu.make_async_copy(
            wts_hbm.at[pl.ds(bag0 * _KP, _C * _KP)], w_v[slot], sw[slot])
        ci.start()
        cw.start()
        if sync:
          ci.wait()
          cw.wait()

      def stage_wait(slot):
        pltpu.make_async_copy(
            ids_hbm.at[pl.ds(0, _C * _K)], idx_v[slot], si[slot]).wait()
        pltpu.make_async_copy(
            wts_hbm.at[pl.ds(0, _C * _KP)], w_v[slot], sw[slot]).wait()

      def gather(s4):
        half = _C * _K // 2
        pltpu.make_async_copy(
            table_hbm.at[idx_v[s4].at[pl.ds(0, half)]],
            rows_v[s4].at[pl.ds(0, half)], sg[s4]).start()
        pltpu.make_async_copy(
            table_hbm.at[idx_v[s4].at[pl.ds(half, half)]],
            rows_v[s4].at[pl.ds(half, half)], sg[s4]).start()

      def gather_wait(s4):
        half = _C * _K // 2
        pltpu.make_async_copy(
            table_hbm.at[idx_v[s4].at[pl.ds(0, half)]],
            rows_v[s4].at[pl.ds(0, half)], sg[s4]).wait()
        pltpu.make_async_copy(
            table_hbm.at[idx_v[s4].at[pl.ds(half, half)]],
            rows_v[s4].at[pl.ds(half, half)], sg[s4]).wait()

      def out_start(it, slot):
        bag0 = base + it * _C
        pltpu.make_async_copy(
            out_v[slot], out_hbm.at[pl.ds(bag0, _C)], so[slot]).start()

      def out_wait(slot):
        pltpu.make_async_copy(
            out_v[slot], out_hbm.at[pl.ds(base, _C)], so[slot]).wait()

      def compute(slot, oslot):
        def bag(i, carry):
          wv0 = w_v[slot][pl.ds(i * _KP, _L)]
          wv1 = w_v[slot][pl.ds(i * _KP + _L, _L)]
          accs = [jnp.zeros((_L,), jnp.float32) for _ in range(_H // _L)]
          for j in range(_K):
            w = wv0[j] if j < _L else wv1[j - _L]
            for h in range(_H // _L):
              accs[h] = accs[h] + w * rows_v[slot][i * _K + j,
                                                   pl.ds(h * _L, _L)]
          for h in range(_H // _L):
            out_v[oslot][i, pl.ds(h * _L, _L)] = accs[h]
          return carry

        lax.fori_loop(0, _C, bag, 0)

      stage(0, 0, sync=True)
      gather(0)
      stage(1, 1, sync=True)
      gather(1)
      stage(2, 2)

      def step(it4, carry):
        for b in range(4):
          it = it4 * 4 + b
          s2 = b % 2

          @pl.when(it < iters - 2)
          def _fire_next():
            stage_wait((b + 2) % 4)
            gather((b + 2) % 4)

          gather_wait(b)

          @pl.when(it < iters - 3)
          def _stage_ahead():
            stage(it + 3, (b + 3) % 4)

          @pl.when(it >= 2)
          def _drain_out():
            out_wait(s2)

          compute(b, s2)
          out_start(it, s2)
        return carry

      lax.fori_loop(0, iters // 4, step, 0)
      out_wait(0)
      out_wait(1)

    run_phase(sids_hbm, sw_hbm, outs_hbm, b_static // _NW)
    run_phase(dids_hbm, dw_hbm, outd_hbm, b_dynamic // _NW)

  return sc_embed


def kernel(static_ids, static_values, static_values_mask, dynamic_ids,
           dynamic_values, dynamic_values_mask, event_mask, table):
  b, ns = static_ids.shape
  bd, s, m = dynamic_ids.shape
  v, h = table.shape
  assert ns == _K and m == _K and h == _H

  sw = jnp.where(static_values_mask, static_values, 1.0)
  sw = sw * (static_ids != 0).astype(jnp.float32)
  dw = jnp.where(dynamic_values_mask, dynamic_values, 1.0)
  dw = dw * (dynamic_ids != 0).astype(jnp.float32)
  dw = dw * event_mask[:, :, None].astype(jnp.float32)
  sw = jnp.pad(sw, ((0, 0), (0, _KP - _K)))
  dw = jnp.pad(dw, ((0, 0), (0, 0), (0, _KP - _K)))

  out_s, out_d = _make_sc_embed(b, bd * s, v)(
      static_ids.astype(jnp.int32).reshape(-1),
      sw.reshape(-1),
      dynamic_ids.astype(jnp.int32).reshape(-1),
      dw.reshape(-1),
      table)
  return (out_s, out_d.reshape(bd, s, _H))
